# baseline (speedup 1.0000x reference)
import functools

import jax
import jax.numpy as jnp
from jax import lax
from jax.experimental import pallas as pl
from jax.experimental.pallas import tpu as pltpu
from jax.experimental.pallas import tpu_sc as plsc

_BN = 400
_NC = 2
_NS = 16
_NW = _NC * _NS
_B = 10240
_BPW = _B // _NW
_CHUNK = 64
_NCHUNK = _BPW // _CHUNK
_SORTN = 16384


def _mesh():
    return plsc.VectorSubcoreMesh(
        core_axis_name="c", subcore_axis_name="s",
        num_cores=_NC, num_subcores=_NS)


def _head_body(x_ref, w_ref, b_ref, asd_ref, h_ref, u_ref, v_ref):
    h = jnp.dot(x_ref[...], w_ref[...], preferred_element_type=jnp.float32)
    h = h + b_ref[...]
    fout = h.shape[1]
    h_ref[:, 0:fout] = h
    if fout < 128:
        h_ref[:, fout:] = jnp.zeros((_BN, 128 - fout), jnp.float32)
    uv = jnp.dot(h, asd_ref[...], preferred_element_type=jnp.float32)
    u_ref[...] = uv[:, 0:1]
    v_ref[...] = uv[:, 1:2]


def _head(x, W, b, asd):
    n, fin = x.shape
    fout = W.shape[1]
    return pl.pallas_call(
        _head_body,
        grid=(n // _BN,),
        in_specs=[
            pl.BlockSpec((_BN, fin), lambda i: (i, 0)),
            pl.BlockSpec((fin, fout), lambda i: (0, 0)),
            pl.BlockSpec((1, fout), lambda i: (0, 0)),
            pl.BlockSpec((fout, 2), lambda i: (0, 0)),
        ],
        out_specs=[
            pl.BlockSpec((_BN, 128), lambda i: (i, 0)),
            pl.BlockSpec((_BN, 1), lambda i: (i, 0)),
            pl.BlockSpec((_BN, 1), lambda i: (i, 0)),
        ],
        out_shape=[
            jax.ShapeDtypeStruct((n, 128), jnp.float32),
            jax.ShapeDtypeStruct((n, 1), jnp.float32),
            jax.ShapeDtypeStruct((n, 1), jnp.float32),
        ],
    )(x, W, b, asd)


def _sc_gather_rows(table, idx):
    v, d = table.shape

    @functools.partial(
        pl.kernel,
        mesh=_mesh(),
        out_type=jax.ShapeDtypeStruct((_B, d), jnp.float32),
        scratch_types=[
            pltpu.VMEM((_BPW,), jnp.int32),
            pltpu.VMEM((_BPW, d), jnp.float32),
            pltpu.SemaphoreType.DMA,
        ],
        compiler_params=pltpu.CompilerParams(needs_layout_passes=False),
    )
    def k(table_hbm, idx_hbm, out_hbm, idx_v, rows_v, sem):
        wid = lax.axis_index("s") * _NC + lax.axis_index("c")
        base = wid * _BPW
        pltpu.sync_copy(idx_hbm.at[pl.ds(base, _BPW)], idx_v)
        for c in range(_NCHUNK):
            pltpu.async_copy(
                table_hbm.at[idx_v.at[pl.ds(c * _CHUNK, _CHUNK)]],
                rows_v.at[pl.ds(c * _CHUNK, _CHUNK)], sem).wait()
        pltpu.sync_copy(rows_v, out_hbm.at[pl.ds(base, _BPW)])

    return k(table, idx)


def _sc_search(vs_pad, theta):
    @functools.partial(
        pl.kernel,
        mesh=_mesh(),
        out_type=(
            jax.ShapeDtypeStruct((_B,), jnp.int32),
            jax.ShapeDtypeStruct((_B,), jnp.int32),
        ),
        scratch_types=[
            pltpu.VMEM((_B,), jnp.float32),
            pltpu.VMEM((_BPW,), jnp.float32),
            pltpu.VMEM((_BPW,), jnp.int32),
            pltpu.VMEM((_BPW,), jnp.int32),
        ],
        compiler_params=pltpu.CompilerParams(needs_layout_passes=False),
    )
    def k(vs_hbm, th_hbm, k_hbm, idx_hbm, vs_v, th_v, idx_v, k_v):
        wid = lax.axis_index("s") * _NC + lax.axis_index("c")
        base = wid * _BPW
        pltpu.sync_copy(vs_hbm, vs_v)
        pltpu.sync_copy(th_hbm.at[pl.ds(base, _BPW)], th_v)

        def qbody(qi, _):
            th = th_v[pl.ds(qi * 16, 16)]
            lo = jnp.zeros((16,), jnp.int32)
            hi = jnp.full((16,), _B, jnp.int32)

            def step(_, carry):
                lo, hi = carry
                mid = lax.div(lo + hi, 2)
                vals = plsc.load_gather(vs_v, [mid])
                pred = vals < th
                return jnp.where(pred, mid + 1, lo), jnp.where(pred, hi, mid)

            lo, hi = lax.fori_loop(0, 14, step, (lo, hi))
            k_v[pl.ds(qi * 16, 16)] = lo
            idx_v[pl.ds(qi * 16, 16)] = jnp.maximum(lo - 1, 0)
            return 0

        lax.fori_loop(0, _BPW // 16, qbody, 0)
        pltpu.sync_copy(k_v, k_hbm.at[pl.ds(base, _BPW)])
        pltpu.sync_copy(idx_v, idx_hbm.at[pl.ds(base, _BPW)])

    return k(vs_pad, theta)


def _scan_body(vmax_ref, vs_ref, hs_ref, cat_ref, carry_pq, carry_ph, carry_qh):
    i = pl.program_id(0)

    @pl.when(i == 0)
    def _():
        carry_pq[...] = jnp.zeros_like(carry_pq)
        carry_ph[...] = jnp.zeros_like(carry_ph)
        carry_qh[...] = jnp.zeros_like(carry_qh)

    vmax = vmax_ref[0, 0]
    v = vs_ref[...]
    p = jnp.exp(v - vmax)
    q = jnp.exp(0.2 * (v - vmax))
    r = lax.broadcasted_iota(jnp.int32, (_BN, _BN), 0)
    c = lax.broadcasted_iota(jnp.int32, (_BN, _BN), 1)
    tril = (c <= r).astype(jnp.float32)
    f = carry_ph.shape[1]
    h = hs_ref[:, 0:f]
    pq = jnp.concatenate([p, q], axis=1)
    cpq = jnp.dot(tril, pq, preferred_element_type=jnp.float32) + carry_pq[...]
    cph = jnp.dot(tril, p * h, preferred_element_type=jnp.float32) + carry_ph[...]
    cqh = jnp.dot(tril, q * h, preferred_element_type=jnp.float32) + carry_qh[...]
    d = _catw(f)
    rep = cat_ref.shape[1] // d
    for r in range(rep):
        o = r * d
        cat_ref[:, o:o + f] = cph
        cat_ref[:, o + f:o + 2 * f] = cqh
        cat_ref[:, o + 2 * f:o + 2 * f + 2] = cpq
        cat_ref[:, o + 2 * f + 2:o + d] = jnp.zeros(
            (_BN, d - 2 * f - 2), jnp.float32)
    carry_pq[...] = cpq[_BN - 1:, :]
    carry_ph[...] = cph[_BN - 1:, :]
    carry_qh[...] = cqh[_BN - 1:, :]


def _catw(f):
    return ((2 * f + 2 + 127) // 128) * 128


def _scan(vmax, vs, hs_pad, f, rep=1):
    n = vs.shape[0]
    d = _catw(f) * rep
    return pl.pallas_call(
        _scan_body,
        grid=(n // _BN,),
        in_specs=[
            pl.BlockSpec((1, 1), lambda i: (0, 0)),
            pl.BlockSpec((_BN, 1), lambda i: (i, 0)),
            pl.BlockSpec((_BN, 128), lambda i: (i, 0)),
        ],
        out_specs=pl.BlockSpec((_BN, d), lambda i: (i, 0)),
        out_shape=jax.ShapeDtypeStruct((n, d), jnp.float32),
        scratch_shapes=[
            pltpu.VMEM((1, 2), jnp.float32),
            pltpu.VMEM((1, f), jnp.float32),
            pltpu.VMEM((1, f), jnp.float32),
        ],
        compiler_params=pltpu.CompilerParams(
            dimension_semantics=("arbitrary",)),
    )(vmax, vs, hs_pad)


_EB = 512


def _expand_block(g, newr, carry, i):
    ir = lax.broadcasted_iota(jnp.int32, (_EB, _EB), 0)
    ic = lax.broadcasted_iota(jnp.int32, (_EB, _EB), 1)
    cand = jnp.where((ic <= ir) & (newr > 0.5), ic, -1)
    lastnew = jnp.max(cand, axis=1, keepdims=True)
    sel = (ic == lastnew).astype(jnp.float32)
    got = jnp.dot(sel, g, preferred_element_type=jnp.float32)
    has = (lastnew >= 0).astype(jnp.float32)
    out = got + (1.0 - has) * carry[...]
    carry[...] = out[_EB - 1:, :]
    return out


def _combine_math(vmax_ref, u_ref, k_ref, tot_ref, gcat, f):
    vmax = vmax_ref[0, 0]
    t = 0.8 * (u_ref[...] + vmax)
    m = jnp.maximum(t, 0.0)
    e1 = jnp.exp(t - m)
    e2 = jnp.exp(-m)
    nz = (k_ref[...] > 0).astype(jnp.float32)
    gph = gcat[:, 0:f]
    gqh = gcat[:, f:2 * f]
    gp = gcat[:, 2 * f:2 * f + 1] * nz
    gq = gcat[:, 2 * f + 1:2 * f + 2] * nz
    tph = tot_ref[:, 0:f]
    tp = tot_ref[0, 2 * f]
    num = e1 * (tph - gph * nz) + e2 * (gqh * nz)
    den = e1 * (tp - gp) + e2 * gq
    return jnp.maximum(num / den, 0.0)


def _combine_body(vmax_ref, u_ref, k_ref, tot_ref, g_ref, newr_ref,
                  out_ref, carry):
    i = pl.program_id(0)

    @pl.when(i == 0)
    def _():
        carry[...] = jnp.zeros_like(carry)

    f = out_ref.shape[1]
    gcat = _expand_block(g_ref[...], newr_ref[...], carry, i)
    out_ref[...] = _combine_math(vmax_ref, u_ref, k_ref, tot_ref, gcat, f)


def _combine_cls_body(vmax_ref, u_ref, k_ref, tot_ref, g_ref, newr_ref,
                      wc_ref, bc_ref, out_ref, carry):
    i = pl.program_id(0)

    @pl.when(i == 0)
    def _():
        carry[...] = jnp.zeros_like(carry)

    f = wc_ref.shape[0]
    gcat = _expand_block(g_ref[...], newr_ref[...], carry, i)
    h3 = _combine_math(vmax_ref, u_ref, k_ref, tot_ref, gcat, f)
    logit = jnp.dot(h3, wc_ref[...], preferred_element_type=jnp.float32)
    out_ref[...] = jax.nn.sigmoid(logit + bc_ref[...])


def _make_combine_head_body(emit_h):
    def body(vmax_ref, u_ref, k_ref, tot_ref, g_ref, newr_ref,
             w_ref, b_ref, asd_ref, *rest):
        i = pl.program_id(0)
        if emit_h:
            hcur_ref, h_ref, u2_ref, v2_ref, carry = rest
        else:
            h_ref, u2_ref, v2_ref, carry = rest

        @pl.when(i == 0)
        def _():
            carry[...] = jnp.zeros_like(carry)

        f = w_ref.shape[0]
        fn = w_ref.shape[1]
        gcat = _expand_block(g_ref[...], newr_ref[...], carry, i)
        hcur = _combine_math(vmax_ref, u_ref, k_ref, tot_ref, gcat, f)
        if emit_h:
            hcur_ref[...] = hcur
        hn = jnp.dot(hcur, w_ref[...], preferred_element_type=jnp.float32)
        hn = hn + b_ref[...]
        h_ref[:, 0:fn] = hn
        if fn < 128:
            h_ref[:, fn:] = jnp.zeros((_EB, 128 - fn), jnp.float32)
        uv = jnp.dot(hn, asd_ref[...], preferred_element_type=jnp.float32)
        u2_ref[...] = uv[:, 0:1]
        v2_ref[...] = uv[:, 1:2]
    return body


def _combine(n, f, vmax, u, kk, tot, g0, newr, wc=None, bc=None,
             nxt=None, emit_h=False):
    d = _catw(f)
    in_specs = [
        pl.BlockSpec((1, 1), lambda i: (0, 0)),
        pl.BlockSpec((_EB, 1), lambda i: (i, 0)),
        pl.BlockSpec((_EB, 1), lambda i: (i, 0)),
        pl.BlockSpec((1, d), lambda i: (0, 0)),
        pl.BlockSpec((_EB, d), lambda i: (i, 0)),
        pl.BlockSpec((1, _EB), lambda i: (0, i)),
    ]
    args = [vmax, u, kk, tot, g0, newr]
    if nxt is not None:
        wn, bn, asdn = nxt
        fn = wn.shape[1]
        body = _make_combine_head_body(emit_h)
        in_specs += [
            pl.BlockSpec((f, fn), lambda i: (0, 0)),
            pl.BlockSpec((1, fn), lambda i: (0, 0)),
            pl.BlockSpec((fn, 2), lambda i: (0, 0)),
        ]
        args += [wn, bn, asdn]
        out_specs = []
        out_shape = []
        if emit_h:
            out_specs.append(pl.BlockSpec((_EB, f), lambda i: (i, 0)))
            out_shape.append(jax.ShapeDtypeStruct((n, f), jnp.float32))
        out_specs += [
            pl.BlockSpec((_EB, 128), lambda i: (i, 0)),
            pl.BlockSpec((_EB, 1), lambda i: (i, 0)),
            pl.BlockSpec((_EB, 1), lambda i: (i, 0)),
        ]
        out_shape += [
            jax.ShapeDtypeStruct((n, 128), jnp.float32),
            jax.ShapeDtypeStruct((n, 1), jnp.float32),
            jax.ShapeDtypeStruct((n, 1), jnp.float32),
        ]
    elif wc is None:
        body = _combine_body
        out_dim = f
        out_specs = pl.BlockSpec((_EB, out_dim), lambda i: (i, 0))
        out_shape = jax.ShapeDtypeStruct((n, out_dim), jnp.float32)
    else:
        body = _combine_cls_body
        out_dim = wc.shape[1]
        in_specs += [
            pl.BlockSpec((f, out_dim), lambda i: (0, 0)),
            pl.BlockSpec((1, out_dim), lambda i: (0, 0)),
        ]
        args += [wc, bc]
        out_specs = pl.BlockSpec((_EB, out_dim), lambda i: (i, 0))
        out_shape = jax.ShapeDtypeStruct((n, out_dim), jnp.float32)
    return pl.pallas_call(
        body,
        grid=(_B // _EB,),
        in_specs=in_specs,
        out_specs=out_specs,
        out_shape=out_shape,
        scratch_shapes=[pltpu.VMEM((1, d), jnp.float32)],
        compiler_params=pltpu.CompilerParams(
            dimension_semantics=("arbitrary",)),
    )(*args)


def _layer_core(h, u, v, f, wc=None, bc=None, nxt=None, emit_h=False):
    n = u.shape[0]
    pad = _B - n
    iv = lax.bitcast_convert_type(v.reshape(n), jnp.int32)
    key = iv ^ ((iv >> 31) & jnp.int32(0x7FFFFFFF))
    spad = _SORTN - n
    keyp = jnp.concatenate([key, jnp.full((spad,), jnp.int32(0x7FFFFFFF))])
    ks, perm_all = lax.sort_key_val(keyp, jnp.arange(_SORTN, dtype=jnp.int32))
    ks = ks[:n]
    perm = perm_all[:n]
    vs = lax.bitcast_convert_type(ks ^ ((ks >> 31) & jnp.int32(0x7FFFFFFF)),
                                  jnp.float32)
    vs_pad = jnp.concatenate([vs, jnp.full((pad,), jnp.inf, jnp.float32)])
    perm_pad = jnp.concatenate([perm, jnp.zeros((pad,), jnp.int32)])
    theta = jnp.concatenate([-u.reshape(n), jnp.zeros((pad,), jnp.float32)])
    hs_pad = _sc_gather_rows(h, perm_pad)
    vmax = vs[n - 1:].reshape(1, 1)
    rep = 4 if f == 64 else 1
    cat = _scan(vmax, vs.reshape(n, 1), hs_pad, f, rep)
    tot = cat[n - 1:, 0:_catw(f)]
    kk, km1 = _sc_search(vs_pad, theta)
    new = jnp.concatenate(
        [jnp.ones((1,), jnp.bool_), km1[1:] != km1[:-1]])
    fill = jnp.arange(_B, dtype=jnp.int32) % n
    if rep > 1:
        table = cat.reshape(n * rep, _catw(f))
        spread = km1 * rep + (jnp.arange(_B, dtype=jnp.int32) % rep)
        gidx = jnp.where(new, spread, fill)
        g0 = _sc_gather_rows(table, gidx)
    else:
        gidx = jnp.where(new, km1, fill)
        g0 = _sc_gather_rows(cat, gidx)
    newf = new.astype(jnp.float32)
    return _combine(n, f, vmax, u, kk.reshape(_B, 1), tot,
                    g0, newf.reshape(1, _B), wc, bc, nxt, emit_h)


@jax.jit
def kernel(x, W1, b1, a1s, a1d, W2, b2, a2s, a2d, W3, b3, a3s, a3d, Wc, bc):
    asd1 = jnp.stack([a1s, a1d], axis=1)
    asd2 = jnp.stack([a2s, a2d], axis=1)
    asd3 = jnp.stack([a3s, a3d], axis=1)
    h, u, v = _head(x, W1, b1.reshape(1, -1), asd1)
    h1, h2t, u2, v2 = _layer_core(
        h, u, v, 128, nxt=(W2, b2.reshape(1, -1), asd2), emit_h=True)
    h3t, u3, v3 = _layer_core(h2t, u2, v2, 64, nxt=(W3, b3.reshape(1, -1), asd3))
    scores = _layer_core(h3t, u3, v3, 32, wc=Wc, bc=bc.reshape(1, 1))
    return (scores, h1)


# speedup vs baseline: 1.0777x; 1.0777x over previous
import functools

import jax
import jax.numpy as jnp
from jax import lax
from jax.experimental import pallas as pl
from jax.experimental.pallas import tpu as pltpu
from jax.experimental.pallas import tpu_sc as plsc

_BN = 1000
_NC = 2
_NS = 16
_NW = _NC * _NS
_B = 10240
_BPW = _B // _NW
_CHUNK = 64
_NCHUNK = _BPW // _CHUNK
_SORTN = 16384


def _mesh():
    return plsc.VectorSubcoreMesh(
        core_axis_name="c", subcore_axis_name="s",
        num_cores=_NC, num_subcores=_NS)


def _head_body(x_ref, w_ref, b_ref, asd_ref, h_ref, u_ref, v_ref):
    h = jnp.dot(x_ref[...], w_ref[...], preferred_element_type=jnp.float32)
    h = h + b_ref[...]
    fout = h.shape[1]
    h_ref[:, 0:fout] = h
    if fout < 128:
        h_ref[:, fout:] = jnp.zeros((_BN, 128 - fout), jnp.float32)
    uv = jnp.dot(h, asd_ref[...], preferred_element_type=jnp.float32)
    u_ref[...] = uv[:, 0:1]
    v_ref[...] = uv[:, 1:2]


def _head(x, W, b, asd):
    n, fin = x.shape
    fout = W.shape[1]
    return pl.pallas_call(
        _head_body,
        grid=(n // _BN,),
        in_specs=[
            pl.BlockSpec((_BN, fin), lambda i: (i, 0)),
            pl.BlockSpec((fin, fout), lambda i: (0, 0)),
            pl.BlockSpec((1, fout), lambda i: (0, 0)),
            pl.BlockSpec((fout, 2), lambda i: (0, 0)),
        ],
        out_specs=[
            pl.BlockSpec((_BN, 128), lambda i: (i, 0)),
            pl.BlockSpec((_BN, 1), lambda i: (i, 0)),
            pl.BlockSpec((_BN, 1), lambda i: (i, 0)),
        ],
        out_shape=[
            jax.ShapeDtypeStruct((n, 128), jnp.float32),
            jax.ShapeDtypeStruct((n, 1), jnp.float32),
            jax.ShapeDtypeStruct((n, 1), jnp.float32),
        ],
    )(x, W, b, asd)


def _sc_gather_rows(table, idx):
    v, d = table.shape

    @functools.partial(
        pl.kernel,
        mesh=_mesh(),
        out_type=jax.ShapeDtypeStruct((_B, d), jnp.float32),
        scratch_types=[
            pltpu.VMEM((_BPW,), jnp.int32),
            pltpu.VMEM((_BPW, d), jnp.float32),
            pltpu.SemaphoreType.DMA,
        ],
        compiler_params=pltpu.CompilerParams(needs_layout_passes=False),
    )
    def k(table_hbm, idx_hbm, out_hbm, idx_v, rows_v, sem):
        wid = lax.axis_index("s") * _NC + lax.axis_index("c")
        base = wid * _BPW
        pltpu.sync_copy(idx_hbm.at[pl.ds(base, _BPW)], idx_v)
        for c in range(_NCHUNK):
            pltpu.async_copy(
                table_hbm.at[idx_v.at[pl.ds(c * _CHUNK, _CHUNK)]],
                rows_v.at[pl.ds(c * _CHUNK, _CHUNK)], sem).wait()
        pltpu.sync_copy(rows_v, out_hbm.at[pl.ds(base, _BPW)])

    return k(table, idx)


def _sc_search(vs_pad, theta):
    @functools.partial(
        pl.kernel,
        mesh=_mesh(),
        out_type=(
            jax.ShapeDtypeStruct((_B,), jnp.int32),
            jax.ShapeDtypeStruct((_B,), jnp.int32),
        ),
        scratch_types=[
            pltpu.VMEM((_B,), jnp.float32),
            pltpu.VMEM((_BPW,), jnp.float32),
            pltpu.VMEM((_BPW,), jnp.int32),
            pltpu.VMEM((_BPW,), jnp.int32),
        ],
        compiler_params=pltpu.CompilerParams(needs_layout_passes=False),
    )
    def k(vs_hbm, th_hbm, k_hbm, idx_hbm, vs_v, th_v, idx_v, k_v):
        wid = lax.axis_index("s") * _NC + lax.axis_index("c")
        base = wid * _BPW
        pltpu.sync_copy(vs_hbm, vs_v)
        pltpu.sync_copy(th_hbm.at[pl.ds(base, _BPW)], th_v)

        def qbody(qi, _):
            th = th_v[pl.ds(qi * 16, 16)]
            lo = jnp.zeros((16,), jnp.int32)
            hi = jnp.full((16,), _B, jnp.int32)

            def step(_, carry):
                lo, hi = carry
                mid = lax.div(lo + hi, 2)
                vals = plsc.load_gather(vs_v, [mid])
                pred = vals < th
                return jnp.where(pred, mid + 1, lo), jnp.where(pred, hi, mid)

            lo, hi = lax.fori_loop(0, 14, step, (lo, hi))
            k_v[pl.ds(qi * 16, 16)] = lo
            idx_v[pl.ds(qi * 16, 16)] = jnp.maximum(lo - 1, 0)
            return 0

        lax.fori_loop(0, _BPW // 16, qbody, 0)
        pltpu.sync_copy(k_v, k_hbm.at[pl.ds(base, _BPW)])
        pltpu.sync_copy(idx_v, idx_hbm.at[pl.ds(base, _BPW)])

    return k(vs_pad, theta)


def _scan_body(vmax_ref, vs_ref, hs_ref, cat_ref, carry_pq, carry_ph, carry_qh):
    i = pl.program_id(0)

    @pl.when(i == 0)
    def _():
        carry_pq[...] = jnp.zeros_like(carry_pq)
        carry_ph[...] = jnp.zeros_like(carry_ph)
        carry_qh[...] = jnp.zeros_like(carry_qh)

    vmax = vmax_ref[0, 0]
    v = vs_ref[...]
    p = jnp.exp(v - vmax)
    q = jnp.exp(0.2 * (v - vmax))
    r = lax.broadcasted_iota(jnp.int32, (_BN, _BN), 0)
    c = lax.broadcasted_iota(jnp.int32, (_BN, _BN), 1)
    tril = (c <= r).astype(jnp.float32)
    f = carry_ph.shape[1]
    h = hs_ref[:, 0:f]
    pq = jnp.concatenate([p, q], axis=1)
    cpq = jnp.dot(tril, pq, preferred_element_type=jnp.float32) + carry_pq[...]
    cph = jnp.dot(tril, p * h, preferred_element_type=jnp.float32) + carry_ph[...]
    cqh = jnp.dot(tril, q * h, preferred_element_type=jnp.float32) + carry_qh[...]
    d = _catw(f)
    rep = cat_ref.shape[1] // d
    for r in range(rep):
        o = r * d
        cat_ref[:, o:o + f] = cph
        cat_ref[:, o + f:o + 2 * f] = cqh
        cat_ref[:, o + 2 * f:o + 2 * f + 2] = cpq
        cat_ref[:, o + 2 * f + 2:o + d] = jnp.zeros(
            (_BN, d - 2 * f - 2), jnp.float32)
    carry_pq[...] = cpq[_BN - 1:, :]
    carry_ph[...] = cph[_BN - 1:, :]
    carry_qh[...] = cqh[_BN - 1:, :]


def _catw(f):
    return ((2 * f + 2 + 127) // 128) * 128


def _scan(vmax, vs, hs_pad, f, rep=1):
    n = vs.shape[0]
    d = _catw(f) * rep
    return pl.pallas_call(
        _scan_body,
        grid=(n // _BN,),
        in_specs=[
            pl.BlockSpec((1, 1), lambda i: (0, 0)),
            pl.BlockSpec((_BN, 1), lambda i: (i, 0)),
            pl.BlockSpec((_BN, 128), lambda i: (i, 0)),
        ],
        out_specs=pl.BlockSpec((_BN, d), lambda i: (i, 0)),
        out_shape=jax.ShapeDtypeStruct((n, d), jnp.float32),
        scratch_shapes=[
            pltpu.VMEM((1, 2), jnp.float32),
            pltpu.VMEM((1, f), jnp.float32),
            pltpu.VMEM((1, f), jnp.float32),
        ],
        compiler_params=pltpu.CompilerParams(
            dimension_semantics=("arbitrary",)),
    )(vmax, vs, hs_pad)


_EB = 1024


def _expand_block(g, newr, carry, i):
    ir = lax.broadcasted_iota(jnp.int32, (_EB, _EB), 0)
    ic = lax.broadcasted_iota(jnp.int32, (_EB, _EB), 1)
    cand = jnp.where((ic <= ir) & (newr > 0.5), ic, -1)
    lastnew = jnp.max(cand, axis=1, keepdims=True)
    sel = (ic == lastnew).astype(jnp.float32)
    got = jnp.dot(sel, g, preferred_element_type=jnp.float32)
    has = (lastnew >= 0).astype(jnp.float32)
    out = got + (1.0 - has) * carry[...]
    carry[...] = out[_EB - 1:, :]
    return out


def _combine_math(vmax_ref, u_ref, k_ref, tot_ref, gcat, f):
    vmax = vmax_ref[0, 0]
    t = 0.8 * (u_ref[...] + vmax)
    m = jnp.maximum(t, 0.0)
    e1 = jnp.exp(t - m)
    e2 = jnp.exp(-m)
    nz = (k_ref[...] > 0).astype(jnp.float32)
    gph = gcat[:, 0:f]
    gqh = gcat[:, f:2 * f]
    gp = gcat[:, 2 * f:2 * f + 1] * nz
    gq = gcat[:, 2 * f + 1:2 * f + 2] * nz
    tph = tot_ref[:, 0:f]
    tp = tot_ref[0, 2 * f]
    num = e1 * (tph - gph * nz) + e2 * (gqh * nz)
    den = e1 * (tp - gp) + e2 * gq
    return jnp.maximum(num / den, 0.0)


def _combine_body(vmax_ref, u_ref, k_ref, tot_ref, g_ref, newr_ref,
                  out_ref, carry):
    i = pl.program_id(0)

    @pl.when(i == 0)
    def _():
        carry[...] = jnp.zeros_like(carry)

    f = out_ref.shape[1]
    gcat = _expand_block(g_ref[...], newr_ref[...], carry, i)
    out_ref[...] = _combine_math(vmax_ref, u_ref, k_ref, tot_ref, gcat, f)


def _combine_cls_body(vmax_ref, u_ref, k_ref, tot_ref, g_ref, newr_ref,
                      wc_ref, bc_ref, out_ref, carry):
    i = pl.program_id(0)

    @pl.when(i == 0)
    def _():
        carry[...] = jnp.zeros_like(carry)

    f = wc_ref.shape[0]
    gcat = _expand_block(g_ref[...], newr_ref[...], carry, i)
    h3 = _combine_math(vmax_ref, u_ref, k_ref, tot_ref, gcat, f)
    logit = jnp.dot(h3, wc_ref[...], preferred_element_type=jnp.float32)
    out_ref[...] = jax.nn.sigmoid(logit + bc_ref[...])


def _make_combine_head_body(emit_h):
    def body(vmax_ref, u_ref, k_ref, tot_ref, g_ref, newr_ref,
             w_ref, b_ref, asd_ref, *rest):
        i = pl.program_id(0)
        if emit_h:
            hcur_ref, h_ref, u2_ref, v2_ref, carry = rest
        else:
            h_ref, u2_ref, v2_ref, carry = rest

        @pl.when(i == 0)
        def _():
            carry[...] = jnp.zeros_like(carry)

        f = w_ref.shape[0]
        fn = w_ref.shape[1]
        gcat = _expand_block(g_ref[...], newr_ref[...], carry, i)
        hcur = _combine_math(vmax_ref, u_ref, k_ref, tot_ref, gcat, f)
        if emit_h:
            hcur_ref[...] = hcur
        hn = jnp.dot(hcur, w_ref[...], preferred_element_type=jnp.float32)
        hn = hn + b_ref[...]
        h_ref[:, 0:fn] = hn
        if fn < 128:
            h_ref[:, fn:] = jnp.zeros((_EB, 128 - fn), jnp.float32)
        uv = jnp.dot(hn, asd_ref[...], preferred_element_type=jnp.float32)
        u2_ref[...] = uv[:, 0:1]
        v2_ref[...] = uv[:, 1:2]
    return body


def _combine(n, f, vmax, u, kk, tot, g0, newr, wc=None, bc=None,
             nxt=None, emit_h=False):
    d = _catw(f)
    in_specs = [
        pl.BlockSpec((1, 1), lambda i: (0, 0)),
        pl.BlockSpec((_EB, 1), lambda i: (i, 0)),
        pl.BlockSpec((_EB, 1), lambda i: (i, 0)),
        pl.BlockSpec((1, d), lambda i: (0, 0)),
        pl.BlockSpec((_EB, d), lambda i: (i, 0)),
        pl.BlockSpec((1, _EB), lambda i: (0, i)),
    ]
    args = [vmax, u, kk, tot, g0, newr]
    if nxt is not None:
        wn, bn, asdn = nxt
        fn = wn.shape[1]
        body = _make_combine_head_body(emit_h)
        in_specs += [
            pl.BlockSpec((f, fn), lambda i: (0, 0)),
            pl.BlockSpec((1, fn), lambda i: (0, 0)),
            pl.BlockSpec((fn, 2), lambda i: (0, 0)),
        ]
        args += [wn, bn, asdn]
        out_specs = []
        out_shape = []
        if emit_h:
            out_specs.append(pl.BlockSpec((_EB, f), lambda i: (i, 0)))
            out_shape.append(jax.ShapeDtypeStruct((n, f), jnp.float32))
        out_specs += [
            pl.BlockSpec((_EB, 128), lambda i: (i, 0)),
            pl.BlockSpec((_EB, 1), lambda i: (i, 0)),
            pl.BlockSpec((_EB, 1), lambda i: (i, 0)),
        ]
        out_shape += [
            jax.ShapeDtypeStruct((n, 128), jnp.float32),
            jax.ShapeDtypeStruct((n, 1), jnp.float32),
            jax.ShapeDtypeStruct((n, 1), jnp.float32),
        ]
    elif wc is None:
        body = _combine_body
        out_dim = f
        out_specs = pl.BlockSpec((_EB, out_dim), lambda i: (i, 0))
        out_shape = jax.ShapeDtypeStruct((n, out_dim), jnp.float32)
    else:
        body = _combine_cls_body
        out_dim = wc.shape[1]
        in_specs += [
            pl.BlockSpec((f, out_dim), lambda i: (0, 0)),
            pl.BlockSpec((1, out_dim), lambda i: (0, 0)),
        ]
        args += [wc, bc]
        out_specs = pl.BlockSpec((_EB, out_dim), lambda i: (i, 0))
        out_shape = jax.ShapeDtypeStruct((n, out_dim), jnp.float32)
    return pl.pallas_call(
        body,
        grid=(_B // _EB,),
        in_specs=in_specs,
        out_specs=out_specs,
        out_shape=out_shape,
        scratch_shapes=[pltpu.VMEM((1, d), jnp.float32)],
        compiler_params=pltpu.CompilerParams(
            dimension_semantics=("arbitrary",)),
    )(*args)


def _layer_core(h, u, v, f, wc=None, bc=None, nxt=None, emit_h=False):
    n = u.shape[0]
    pad = _B - n
    iv = lax.bitcast_convert_type(v.reshape(n), jnp.int32)
    key = iv ^ ((iv >> 31) & jnp.int32(0x7FFFFFFF))
    spad = _SORTN - n
    keyp = jnp.concatenate([key, jnp.full((spad,), jnp.int32(0x7FFFFFFF))])
    ks, perm_all = lax.sort_key_val(keyp, jnp.arange(_SORTN, dtype=jnp.int32))
    ks = ks[:n]
    perm = perm_all[:n]
    vs = lax.bitcast_convert_type(ks ^ ((ks >> 31) & jnp.int32(0x7FFFFFFF)),
                                  jnp.float32)
    vs_pad = jnp.concatenate([vs, jnp.full((pad,), jnp.inf, jnp.float32)])
    perm_pad = jnp.concatenate([perm, jnp.zeros((pad,), jnp.int32)])
    theta = jnp.concatenate([-u.reshape(n), jnp.zeros((pad,), jnp.float32)])
    hs_pad = _sc_gather_rows(h, perm_pad)
    vmax = vs[n - 1:].reshape(1, 1)
    rep = 4 if f == 64 else 1
    cat = _scan(vmax, vs.reshape(n, 1), hs_pad, f, rep)
    tot = cat[n - 1:, 0:_catw(f)]
    kk, km1 = _sc_search(vs_pad, theta)
    new = jnp.concatenate(
        [jnp.ones((1,), jnp.bool_), km1[1:] != km1[:-1]])
    fill = jnp.arange(_B, dtype=jnp.int32) % n
    if rep > 1:
        table = cat.reshape(n * rep, _catw(f))
        spread = km1 * rep + (jnp.arange(_B, dtype=jnp.int32) % rep)
        gidx = jnp.where(new, spread, fill)
        g0 = _sc_gather_rows(table, gidx)
    else:
        gidx = jnp.where(new, km1, fill)
        g0 = _sc_gather_rows(cat, gidx)
    newf = new.astype(jnp.float32)
    return _combine(n, f, vmax, u, kk.reshape(_B, 1), tot,
                    g0, newf.reshape(1, _B), wc, bc, nxt, emit_h)


@jax.jit
def kernel(x, W1, b1, a1s, a1d, W2, b2, a2s, a2d, W3, b3, a3s, a3d, Wc, bc):
    asd1 = jnp.stack([a1s, a1d], axis=1)
    asd2 = jnp.stack([a2s, a2d], axis=1)
    asd3 = jnp.stack([a3s, a3d], axis=1)
    h, u, v = _head(x, W1, b1.reshape(1, -1), asd1)
    h1, h2t, u2, v2 = _layer_core(
        h, u, v, 128, nxt=(W2, b2.reshape(1, -1), asd2), emit_h=True)
    h3t, u3, v3 = _layer_core(h2t, u2, v2, 64, nxt=(W3, b3.reshape(1, -1), asd3))
    scores = _layer_core(h3t, u3, v3, 32, wc=Wc, bc=bc.reshape(1, 1))
    return (scores, h1)
